# baseline (speedup 1.0000x reference)
import jax
import jax.numpy as jnp
from jax import lax
from jax.experimental import pallas as pl
from jax.experimental.pallas import tpu as pltpu, tpu_sc as plsc

B, D1, D2 = 32, 64, 8192
K = 64
NT = D2 // 16
MASK7 = 127
ONE_BITS = 0x3F800000


def _stage1_body(x_ref, thr_ref, valid_ref, vmax_ref, imax_ref):
    x = x_ref[0]
    thr = thr_ref[0, 0]
    valid = x >= thr
    xm = jnp.where(valid, x, 0.0)
    vmax = jnp.max(xm, axis=0, keepdims=True)
    hit = xm == vmax
    ids = lax.broadcasted_iota(jnp.int32, xm.shape, 0)
    imax = jnp.min(jnp.where(hit, ids, D1), axis=0, keepdims=True)
    valid_ref[0] = valid
    vmax_ref[0] = vmax
    imax_ref[0] = imax


S1B = 2048


def _stage1(x, threshold):
    thr = threshold.reshape(1, 1)
    return pl.pallas_call(
        _stage1_body,
        grid=(B, D2 // S1B),
        in_specs=[
            pl.BlockSpec((1, D1, S1B), lambda b, j: (b, 0, j)),
            pl.BlockSpec((1, 1), lambda b, j: (0, 0)),
        ],
        out_specs=[
            pl.BlockSpec((1, D1, S1B), lambda b, j: (b, 0, j)),
            pl.BlockSpec((1, 1, S1B), lambda b, j: (b, 0, j)),
            pl.BlockSpec((1, 1, S1B), lambda b, j: (b, 0, j)),
        ],
        out_shape=[
            jax.ShapeDtypeStruct((B, D1, D2), jnp.bool_),
            jax.ShapeDtypeStruct((B, 1, D2), jnp.float32),
            jax.ShapeDtypeStruct((B, 1, D2), jnp.int32),
        ],
    )(x, thr)


def _sc_body(vmax_hbm, imax_hbm, thr_hbm, idx_hbm, cutv_hbm, cutd2_hbm,
             wq_hbm, dflag_hbm,
             vmax_v, imax_v, thr_v, key_a, d2_a, key_b, d2_b,
             hist_a, hist_b, digs_v, pos_v, starts_v, tots_v, outidx_v,
             cutv_v, cutd2_v, wq_v, dflag_v):
    b = lax.axis_index("s") * 2 + lax.axis_index("c")
    pltpu.sync_copy(vmax_hbm.at[pl.ds(b * D2, D2)], vmax_v)
    pltpu.sync_copy(imax_hbm.at[pl.ds(b * D2, D2)], imax_v)
    pltpu.sync_copy(thr_hbm, thr_v)

    lanes = lax.iota(jnp.int32, 16)
    zeros16 = jnp.zeros((16,), jnp.int32)
    ones16 = jnp.ones((16,), jnp.int32)
    base_bits = plsc.bitcast(thr_v[...], jnp.int32)
    span = (ONE_BITS - base_bits) + 1

    CH = D2 // 16

    @plsc.parallel_loop(0, CH, unroll=4, carry=zeros16)
    def z0(t, z0c):
        gi = t * 16 + lanes
        v = plsc.load_gather(vmax_v, [gi])
        r = plsc.load_gather(imax_v, [gi])
        vb = plsc.bitcast(v, jnp.int32)
        nzm = v > 0.0
        zval = jnp.where(nzm, vb - base_bits + 1, 0)
        key = r * (1 << 22) + (span - zval)
        plsc.store_scatter(key_a, [gi], key)
        plsc.store_scatter(d2_a, [gi], gi)
        plsc.store_scatter(wq_v, [gi], zeros16)

        @pl.when(t < 128)
        def _():
            plsc.store_scatter(hist_a, [t * 16 + lanes], zeros16)
        return z0c + plsc.all_reduce_population_count(~nzm)

    def hist_pass(src_key, hist, shift):
        @plsc.parallel_loop(0, CH, unroll=4)
        def _(t):
            gi = lanes * CH + t
            k = plsc.load_gather(src_key, [gi])
            dig = (k >> shift) & MASK7
            plsc.addupdate_scatter(hist, [dig * 16 + lanes], ones16)
            plsc.store_scatter(digs_v, [gi], dig)

    def scan_pass(hist, other, save_layout):
        def sloop(dig, carry):
            hidx = dig * 16 + lanes
            h = plsc.load_gather(hist, [hidx])
            c = plsc.cumsum(h)
            tot = jnp.sum(h)
            plsc.store_scatter(hist, [hidx], carry + (c - h))
            if other is not None:
                plsc.store_scatter(other, [hidx], zeros16)
            if save_layout:
                m0 = lanes == 0
                digs = zeros16 + dig
                plsc.store_scatter(starts_v, [digs], carry, mask=m0)
                plsc.store_scatter(tots_v, [digs], zeros16 + tot, mask=m0)
            return carry + tot
        lax.fori_loop(0, 128, sloop, zeros16)

    def perm_pass(src_key, src_d2, dst_key, dst_d2, hist):
        def posloop(t, c):
            gi = lanes * CH + t
            dig = plsc.load_gather(digs_v, [gi])
            hidx = dig * 16 + lanes
            pos = plsc.load_gather(hist, [hidx])
            plsc.store_scatter(hist, [hidx], pos + 1)
            plsc.store_scatter(pos_v, [gi], pos)
            return c
        lax.fori_loop(0, CH, posloop, 0)

        @plsc.parallel_loop(0, CH, unroll=4)
        def _(t):
            gi = t * 16 + lanes
            k = plsc.load_gather(src_key, [gi])
            d = plsc.load_gather(src_d2, [gi])
            p = plsc.load_gather(pos_v, [gi])
            plsc.store_scatter(dst_key, [p], k)
            plsc.store_scatter(dst_d2, [p], d)

    hist_pass(key_a, hist_a, 0)
    scan_pass(hist_a, hist_b, False)
    perm_pass(key_a, d2_a, key_b, d2_b, hist_a)
    hist_pass(key_b, hist_b, 7)
    scan_pass(hist_b, hist_a, False)
    perm_pass(key_b, d2_b, key_a, d2_a, hist_b)
    hist_pass(key_a, hist_a, 14)
    scan_pass(hist_a, hist_b, False)
    perm_pass(key_a, d2_a, key_b, d2_b, hist_a)
    hist_pass(key_b, hist_b, 21)
    scan_pass(hist_b, None, True)
    perm_pass(key_b, d2_b, key_a, d2_a, hist_b)

    @plsc.parallel_loop(0, D1, unroll=2, carry=zeros16 + (1 << 30))
    def minnz(r, mn):
        rsp = zeros16 + r
        r2 = rsp * 2
        startr = plsc.load_gather(starts_v, [r2])
        cnt = plsc.load_gather(tots_v, [r2]) + plsc.load_gather(tots_v, [r2 + 1])
        nz = jnp.where(rsp == 0, cnt - z0, cnt)

        for kk in range(4):
            posrow = kk * 16 + lanes
            sel = posrow < nz
            gidx = jnp.minimum(startr + posrow, D2 - 1)
            dv = plsc.load_gather(d2_a, [gidx])
            plsc.store_scatter(outidx_v, [r * K + posrow], jnp.where(sel, dv, 0))
            plsc.store_scatter(wq_v, [jnp.where(sel, dv, 0)], ones16, mask=sel)

        g63 = jnp.minimum(startr + (K - 1), D2 - 1)
        kv63 = plsc.load_gather(key_a, [g63])
        dv63 = plsc.load_gather(d2_a, [g63])
        sel63 = nz > (K - 1)
        zval63 = span - (kv63 & ((1 << 22) - 1))
        v63 = jnp.where(sel63 & (zval63 > 0),
                        plsc.bitcast(zval63 - 1 + base_bits, jnp.float32), 0.0)
        m0 = lanes == 0
        plsc.store_scatter(cutv_v, [rsp], v63, mask=m0)
        plsc.store_scatter(cutd2_v, [rsp], jnp.where(sel63, dv63, 0), mask=m0)
        return jnp.minimum(mn, nz)

    @pl.when(jnp.min(minnz) < K)
    def _pad():
        def fixr(r, c):
            rsp = zeros16 + r
            r2 = rsp * 2
            cnt = plsc.load_gather(tots_v, [r2]) + plsc.load_gather(tots_v, [r2 + 1])
            nz = jnp.where(rsp == 0, cnt - z0, cnt)

            @pl.when(jnp.min(nz) < K)
            def _fix():
                def scan_t(t, cntz):
                    gi = t * 16 + lanes
                    v = plsc.load_gather(vmax_v, [gi])
                    rr = plsc.load_gather(imax_v, [gi])
                    isz = (rr != rsp) | (v <= 0.0)
                    pref = plsc.cumsum(isz.astype(jnp.int32))
                    slot = nz + cntz + pref - 1
                    selp = isz & (slot >= nz) & (slot < K)
                    slot_c = jnp.clip(slot, 0, K - 1)
                    plsc.store_scatter(outidx_v, [r * K + slot_c], gi, mask=selp)
                    return cntz + plsc.all_reduce_population_count(isz)
                lax.fori_loop(0, NT, scan_t, zeros16)
                last = plsc.load_gather(outidx_v, [zeros16 + (r * K + K - 1)])
                plsc.store_scatter(cutd2_v, [rsp], last, mask=lanes == 0)
            return c
        lax.fori_loop(0, D1, fixr, 0)

    pltpu.sync_copy(outidx_v, idx_hbm.at[pl.ds(b * (D1 * K), D1 * K)])
    pltpu.sync_copy(cutv_v, cutv_hbm.at[pl.ds(b * D1, D1)])
    pltpu.sync_copy(cutd2_v, cutd2_hbm.at[pl.ds(b * D1, D1)])
    pltpu.sync_copy(wq_v, wq_hbm.at[pl.ds(b * D2, D2)])
    dflag = jnp.where(jnp.min(minnz) < K, 1, 0) + zeros16
    plsc.store_scatter(dflag_v, [lanes], dflag)
    pltpu.sync_copy(dflag_v, dflag_hbm.at[pl.ds(b * 16, 16)])


def _stage2(vflat, iflat, thr16):
    mesh = plsc.VectorSubcoreMesh(core_axis_name="c", subcore_axis_name="s")
    return pl.kernel(
        _sc_body,
        out_type=[
            jax.ShapeDtypeStruct((B * D1 * K,), jnp.int32),
            jax.ShapeDtypeStruct((B * D1,), jnp.float32),
            jax.ShapeDtypeStruct((B * D1,), jnp.int32),
            jax.ShapeDtypeStruct((B * D2,), jnp.int32),
            jax.ShapeDtypeStruct((B * 16,), jnp.int32),
        ],
        mesh=mesh,
        compiler_params=pltpu.CompilerParams(needs_layout_passes=False),
        scratch_types=[
            pltpu.VMEM((D2,), jnp.float32),
            pltpu.VMEM((D2,), jnp.int32),
            pltpu.VMEM((16,), jnp.float32),
            pltpu.VMEM((D2,), jnp.int32),
            pltpu.VMEM((D2,), jnp.int32),
            pltpu.VMEM((D2,), jnp.int32),
            pltpu.VMEM((D2,), jnp.int32),
            pltpu.VMEM((2048,), jnp.int32),
            pltpu.VMEM((2048,), jnp.int32),
            pltpu.VMEM((D2,), jnp.int32),
            pltpu.VMEM((D2,), jnp.int32),
            pltpu.VMEM((128,), jnp.int32),
            pltpu.VMEM((128,), jnp.int32),
            pltpu.VMEM((D1 * K,), jnp.int32),
            pltpu.VMEM((D1,), jnp.float32),
            pltpu.VMEM((D1,), jnp.int32),
            pltpu.VMEM((D2,), jnp.int32),
            pltpu.VMEM((16,), jnp.int32),
        ],
    )(vflat, iflat, thr16)


S3B = 2048


def _stage3_body(dflag_ref, vmax_ref, imax_ref, cutv_ref, cutd2_ref, wq_ref,
                 feat_ref):
    imax = imax_ref[0]
    wq = wq_ref[0]
    d1ids = lax.broadcasted_iota(jnp.int32, (D1, S3B), 0)
    win = (imax == d1ids) & (wq != 0)

    @pl.when(dflag_ref[0, 0, 0] == 0)
    def _common():
        feat_ref[0] = win.astype(jnp.float32)

    @pl.when(dflag_ref[0, 0, 0] != 0)
    def _deficient():
        vmax = vmax_ref[0]
        cutv = cutv_ref[0]
        cutd2 = cutd2_ref[0]
        d2ids = (lax.broadcasted_iota(jnp.int32, (D1, S3B), 1)
                 + pl.program_id(1) * S3B)
        zpad = (cutv == 0.0) & (d2ids <= cutd2) & ((imax != d1ids) | (vmax == 0.0))
        feat_ref[0] = (win | zpad).astype(jnp.float32)


def _stage3(dflags, vmax3, imax3, cutv, cutd2, wq):
    return pl.pallas_call(
        _stage3_body,
        grid=(B, D2 // S3B),
        in_specs=[
            pl.BlockSpec(memory_space=pltpu.SMEM, block_shape=(1, 1, 1),
                         index_map=lambda b, j: (b, 0, 0)),
            pl.BlockSpec((1, 1, S3B), lambda b, j: (b, 0, j)),
            pl.BlockSpec((1, 1, S3B), lambda b, j: (b, 0, j)),
            pl.BlockSpec((1, D1, 1), lambda b, j: (b, 0, 0)),
            pl.BlockSpec((1, D1, 1), lambda b, j: (b, 0, 0)),
            pl.BlockSpec((1, 1, S3B), lambda b, j: (b, 0, j)),
        ],
        out_specs=pl.BlockSpec((1, D1, S3B), lambda b, j: (b, 0, j)),
        out_shape=jax.ShapeDtypeStruct((B, D1, D2), jnp.float32),
    )(dflags, vmax3, imax3, cutv, cutd2, wq)


def kernel(x, threshold):
    valid, vmax3, imax3 = _stage1(x, threshold)
    thr16 = jnp.full((16,), threshold, dtype=jnp.float32)
    idx_flat, cutv_flat, cutd2_flat, wq_flat, dflag_flat = _stage2(
        vmax3.reshape(-1), imax3.reshape(-1), thr16)
    indices = idx_flat.reshape(B, D1, K)
    dflags = dflag_flat.reshape(B, 16)[:, :1].reshape(B, 1, 1)
    feat = _stage3(dflags, vmax3, imax3,
                   cutv_flat.reshape(B, D1, 1), cutd2_flat.reshape(B, D1, 1),
                   wq_flat.reshape(B, 1, D2))
    return (feat, indices, valid)


# speedup vs baseline: 1.3004x; 1.3004x over previous
import jax
import jax.numpy as jnp
from jax import lax
from jax.experimental import pallas as pl
from jax.experimental.pallas import tpu as pltpu, tpu_sc as plsc

B, D1, D2 = 32, 64, 8192
K = 64
NT = D2 // 16
MASK7 = 127
ONE_BITS = 0x3F800000


def _stage1_body(x_ref, thr_ref, valid_ref, vmax_ref, imax_ref):
    x = x_ref[0]
    thr = thr_ref[0, 0]
    valid = x >= thr
    xm = jnp.where(valid, x, 0.0)
    vmax = jnp.max(xm, axis=0, keepdims=True)
    hit = xm == vmax
    ids = lax.broadcasted_iota(jnp.int32, xm.shape, 0)
    imax = jnp.min(jnp.where(hit, ids, D1), axis=0, keepdims=True)
    valid_ref[0] = valid
    vmax_ref[0] = vmax
    imax_ref[0] = imax


S1B = D2


def _stage1(x, threshold):
    thr = threshold.reshape(1, 1)
    return pl.pallas_call(
        _stage1_body,
        grid=(B, D2 // S1B),
        in_specs=[
            pl.BlockSpec((1, D1, S1B), lambda b, j: (b, 0, j)),
            pl.BlockSpec((1, 1), lambda b, j: (0, 0)),
        ],
        out_specs=[
            pl.BlockSpec((1, D1, S1B), lambda b, j: (b, 0, j)),
            pl.BlockSpec((1, 1, S1B), lambda b, j: (b, 0, j)),
            pl.BlockSpec((1, 1, S1B), lambda b, j: (b, 0, j)),
        ],
        out_shape=[
            jax.ShapeDtypeStruct((B, D1, D2), jnp.bool_),
            jax.ShapeDtypeStruct((B, 1, D2), jnp.float32),
            jax.ShapeDtypeStruct((B, 1, D2), jnp.int32),
        ],
    )(x, thr)


def _sc_body(vmax_hbm, imax_hbm, thr_hbm, idx_hbm, cutv_hbm, cutd2_hbm,
             wq_hbm, dflag_hbm,
             vmax_v, imax_v, thr_v, key_a, d2_a, key_b, d2_b,
             hist_a, hist_b, digs_v, starts_v, tots_v, outidx_v,
             cutv_v, cutd2_v, wq_v, dflag_v):
    b = lax.axis_index("s") * 2 + lax.axis_index("c")
    pltpu.sync_copy(vmax_hbm.at[pl.ds(b * D2, D2)], vmax_v)
    pltpu.sync_copy(imax_hbm.at[pl.ds(b * D2, D2)], imax_v)
    pltpu.sync_copy(thr_hbm, thr_v)

    lanes = lax.iota(jnp.int32, 16)
    zeros16 = jnp.zeros((16,), jnp.int32)
    ones16 = jnp.ones((16,), jnp.int32)
    base_bits = plsc.bitcast(thr_v[...], jnp.int32)
    span = (ONE_BITS - base_bits) + 1

    CH = D2 // 16

    @plsc.parallel_loop(0, CH, unroll=4, carry=zeros16)
    def z0(t, z0c):
        gi = t * 16 + lanes
        v = plsc.load_gather(vmax_v, [gi])
        r = plsc.load_gather(imax_v, [gi])
        vb = plsc.bitcast(v, jnp.int32)
        nzm = v > 0.0
        zval = jnp.where(nzm, vb - base_bits + 1, 0)
        key = r * (1 << 22) + (span - zval)
        plsc.store_scatter(key_a, [gi], key)
        plsc.store_scatter(d2_a, [gi], gi)
        plsc.store_scatter(wq_v, [gi], zeros16)

        @pl.when(t < 128)
        def _():
            plsc.store_scatter(hist_a, [t * 16 + lanes], zeros16)
        return z0c + plsc.all_reduce_population_count(~nzm)

    def hist_pass(src_key, hist, shift):
        @plsc.parallel_loop(0, CH, unroll=4)
        def _(t):
            gi = lanes * CH + t
            k = plsc.load_gather(src_key, [gi])
            dig = (k >> shift) & MASK7
            plsc.addupdate_scatter(hist, [dig * 16 + lanes], ones16)
            plsc.store_scatter(digs_v, [gi], dig)

    def scan_pass(hist, other, save_layout):
        def sloop(dig, carry):
            hidx = dig * 16 + lanes
            h = plsc.load_gather(hist, [hidx])
            c = plsc.cumsum(h)
            tot = jnp.sum(h)
            plsc.store_scatter(hist, [hidx], carry + (c - h))
            if other is not None:
                plsc.store_scatter(other, [hidx], zeros16)
            if save_layout:
                m0 = lanes == 0
                digs = zeros16 + dig
                plsc.store_scatter(starts_v, [digs], carry, mask=m0)
                plsc.store_scatter(tots_v, [digs], zeros16 + tot, mask=m0)
            return carry + tot
        lax.fori_loop(0, 128, sloop, zeros16)

    def perm_pass(src_key, src_d2, dst_key, dst_d2, hist):
        def posloop(t, c):
            gi = lanes * CH + t
            dig = plsc.load_gather(digs_v, [gi])
            hidx = dig * 16 + lanes
            pos = plsc.load_gather(hist, [hidx])
            plsc.store_scatter(hist, [hidx], pos + 1)
            k = plsc.load_gather(src_key, [gi])
            d = plsc.load_gather(src_d2, [gi])
            plsc.store_scatter(dst_key, [pos], k)
            plsc.store_scatter(dst_d2, [pos], d)
            return c
        lax.fori_loop(0, CH, posloop, 0)

    hist_pass(key_a, hist_a, 0)
    scan_pass(hist_a, hist_b, False)
    perm_pass(key_a, d2_a, key_b, d2_b, hist_a)
    hist_pass(key_b, hist_b, 7)
    scan_pass(hist_b, hist_a, False)
    perm_pass(key_b, d2_b, key_a, d2_a, hist_b)
    hist_pass(key_a, hist_a, 14)
    scan_pass(hist_a, hist_b, False)
    perm_pass(key_a, d2_a, key_b, d2_b, hist_a)
    hist_pass(key_b, hist_b, 21)
    scan_pass(hist_b, None, True)
    perm_pass(key_b, d2_b, key_a, d2_a, hist_b)

    @plsc.parallel_loop(0, D1, unroll=2, carry=zeros16 + (1 << 30))
    def minnz(r, mn):
        rsp = zeros16 + r
        r2 = rsp * 2
        startr = plsc.load_gather(starts_v, [r2])
        cnt = plsc.load_gather(tots_v, [r2]) + plsc.load_gather(tots_v, [r2 + 1])
        nz = jnp.where(rsp == 0, cnt - z0, cnt)

        for kk in range(4):
            posrow = kk * 16 + lanes
            sel = posrow < nz
            gidx = jnp.minimum(startr + posrow, D2 - 1)
            dv = plsc.load_gather(d2_a, [gidx])
            plsc.store_scatter(outidx_v, [r * K + posrow], jnp.where(sel, dv, 0))
            plsc.store_scatter(wq_v, [jnp.where(sel, dv, 0)], ones16, mask=sel)

        g63 = jnp.minimum(startr + (K - 1), D2 - 1)
        kv63 = plsc.load_gather(key_a, [g63])
        dv63 = plsc.load_gather(d2_a, [g63])
        sel63 = nz > (K - 1)
        zval63 = span - (kv63 & ((1 << 22) - 1))
        v63 = jnp.where(sel63 & (zval63 > 0),
                        plsc.bitcast(zval63 - 1 + base_bits, jnp.float32), 0.0)
        m0 = lanes == 0
        plsc.store_scatter(cutv_v, [rsp], v63, mask=m0)
        plsc.store_scatter(cutd2_v, [rsp], jnp.where(sel63, dv63, 0), mask=m0)
        return jnp.minimum(mn, nz)

    @pl.when(jnp.min(minnz) < K)
    def _pad():
        def fixr(r, c):
            rsp = zeros16 + r
            r2 = rsp * 2
            cnt = plsc.load_gather(tots_v, [r2]) + plsc.load_gather(tots_v, [r2 + 1])
            nz = jnp.where(rsp == 0, cnt - z0, cnt)

            @pl.when(jnp.min(nz) < K)
            def _fix():
                def scan_t(t, cntz):
                    gi = t * 16 + lanes
                    v = plsc.load_gather(vmax_v, [gi])
                    rr = plsc.load_gather(imax_v, [gi])
                    isz = (rr != rsp) | (v <= 0.0)
                    pref = plsc.cumsum(isz.astype(jnp.int32))
                    slot = nz + cntz + pref - 1
                    selp = isz & (slot >= nz) & (slot < K)
                    slot_c = jnp.clip(slot, 0, K - 1)
                    plsc.store_scatter(outidx_v, [r * K + slot_c], gi, mask=selp)
                    return cntz + plsc.all_reduce_population_count(isz)
                lax.fori_loop(0, NT, scan_t, zeros16)
                last = plsc.load_gather(outidx_v, [zeros16 + (r * K + K - 1)])
                plsc.store_scatter(cutd2_v, [rsp], last, mask=lanes == 0)
            return c
        lax.fori_loop(0, D1, fixr, 0)

    pltpu.sync_copy(outidx_v, idx_hbm.at[pl.ds(b * (D1 * K), D1 * K)])
    pltpu.sync_copy(cutv_v, cutv_hbm.at[pl.ds(b * D1, D1)])
    pltpu.sync_copy(cutd2_v, cutd2_hbm.at[pl.ds(b * D1, D1)])
    pltpu.sync_copy(wq_v, wq_hbm.at[pl.ds(b * D2, D2)])
    dflag = jnp.where(jnp.min(minnz) < K, 1, 0) + zeros16
    plsc.store_scatter(dflag_v, [lanes], dflag)
    pltpu.sync_copy(dflag_v, dflag_hbm.at[pl.ds(b * 16, 16)])


def _stage2(vflat, iflat, thr16):
    mesh = plsc.VectorSubcoreMesh(core_axis_name="c", subcore_axis_name="s")
    return pl.kernel(
        _sc_body,
        out_type=[
            jax.ShapeDtypeStruct((B * D1 * K,), jnp.int32),
            jax.ShapeDtypeStruct((B * D1,), jnp.float32),
            jax.ShapeDtypeStruct((B * D1,), jnp.int32),
            jax.ShapeDtypeStruct((B * D2,), jnp.int32),
            jax.ShapeDtypeStruct((B * 16,), jnp.int32),
        ],
        mesh=mesh,
        compiler_params=pltpu.CompilerParams(needs_layout_passes=False),
        scratch_types=[
            pltpu.VMEM((D2,), jnp.float32),
            pltpu.VMEM((D2,), jnp.int32),
            pltpu.VMEM((16,), jnp.float32),
            pltpu.VMEM((D2,), jnp.int32),
            pltpu.VMEM((D2,), jnp.int32),
            pltpu.VMEM((D2,), jnp.int32),
            pltpu.VMEM((D2,), jnp.int32),
            pltpu.VMEM((2048,), jnp.int32),
            pltpu.VMEM((2048,), jnp.int32),
            pltpu.VMEM((D2,), jnp.int32),
            pltpu.VMEM((128,), jnp.int32),
            pltpu.VMEM((128,), jnp.int32),
            pltpu.VMEM((D1 * K,), jnp.int32),
            pltpu.VMEM((D1,), jnp.float32),
            pltpu.VMEM((D1,), jnp.int32),
            pltpu.VMEM((D2,), jnp.int32),
            pltpu.VMEM((16,), jnp.int32),
        ],
    )(vflat, iflat, thr16)


S3B = D2


def _stage3_body(dflag_ref, vmax_ref, imax_ref, cutv_ref, cutd2_ref, wq_ref,
                 feat_ref):
    imax = imax_ref[0]
    wq = wq_ref[0]
    d1ids = lax.broadcasted_iota(jnp.int32, (D1, S3B), 0)
    win = (imax == d1ids) & (wq != 0)

    @pl.when(dflag_ref[0, 0, 0] == 0)
    def _common():
        feat_ref[0] = win.astype(jnp.float32)

    @pl.when(dflag_ref[0, 0, 0] != 0)
    def _deficient():
        vmax = vmax_ref[0]
        cutv = cutv_ref[0]
        cutd2 = cutd2_ref[0]
        d2ids = (lax.broadcasted_iota(jnp.int32, (D1, S3B), 1)
                 + pl.program_id(1) * S3B)
        zpad = (cutv == 0.0) & (d2ids <= cutd2) & ((imax != d1ids) | (vmax == 0.0))
        feat_ref[0] = (win | zpad).astype(jnp.float32)


def _stage3(dflags, vmax3, imax3, cutv, cutd2, wq):
    return pl.pallas_call(
        _stage3_body,
        grid=(B, D2 // S3B),
        in_specs=[
            pl.BlockSpec(memory_space=pltpu.SMEM, block_shape=(1, 1, 1),
                         index_map=lambda b, j: (b, 0, 0)),
            pl.BlockSpec((1, 1, S3B), lambda b, j: (b, 0, j)),
            pl.BlockSpec((1, 1, S3B), lambda b, j: (b, 0, j)),
            pl.BlockSpec((1, D1, 1), lambda b, j: (b, 0, 0)),
            pl.BlockSpec((1, D1, 1), lambda b, j: (b, 0, 0)),
            pl.BlockSpec((1, 1, S3B), lambda b, j: (b, 0, j)),
        ],
        out_specs=pl.BlockSpec((1, D1, S3B), lambda b, j: (b, 0, j)),
        out_shape=jax.ShapeDtypeStruct((B, D1, D2), jnp.float32),
    )(dflags, vmax3, imax3, cutv, cutd2, wq)


def kernel(x, threshold):
    valid, vmax3, imax3 = _stage1(x, threshold)
    thr16 = jnp.full((16,), threshold, dtype=jnp.float32)
    idx_flat, cutv_flat, cutd2_flat, wq_flat, dflag_flat = _stage2(
        vmax3.reshape(-1), imax3.reshape(-1), thr16)
    indices = idx_flat.reshape(B, D1, K)
    dflags = dflag_flat.reshape(B, 16)[:, :1].reshape(B, 1, 1)
    feat = _stage3(dflags, vmax3, imax3,
                   cutv_flat.reshape(B, D1, 1), cutd2_flat.reshape(B, D1, 1),
                   wq_flat.reshape(B, 1, D2))
    return (feat, indices, valid)


# speedup vs baseline: 1.4314x; 1.1008x over previous
import jax
import jax.numpy as jnp
from jax import lax
from jax.experimental import pallas as pl
from jax.experimental.pallas import tpu as pltpu, tpu_sc as plsc

B, D1, D2 = 32, 64, 8192
K = 64
NT = D2 // 16
MASK7 = 127
ONE_BITS = 0x3F800000


def _stage1_body(x_ref, thr_ref, valid_ref, vmax_ref, imax_ref):
    x = x_ref[0]
    thr = thr_ref[0, 0]
    valid = x >= thr
    xm = jnp.where(valid, x, 0.0)
    vmax = jnp.max(xm, axis=0, keepdims=True)
    hit = xm == vmax
    ids = lax.broadcasted_iota(jnp.int32, xm.shape, 0)
    imax = jnp.min(jnp.where(hit, ids, D1), axis=0, keepdims=True)
    valid_ref[0] = valid
    vmax_ref[0] = vmax
    imax_ref[0] = imax


S1B = D2


def _stage1(x, threshold):
    thr = threshold.reshape(1, 1)
    return pl.pallas_call(
        _stage1_body,
        grid=(B, D2 // S1B),
        in_specs=[
            pl.BlockSpec((1, D1, S1B), lambda b, j: (b, 0, j)),
            pl.BlockSpec((1, 1), lambda b, j: (0, 0)),
        ],
        out_specs=[
            pl.BlockSpec((1, D1, S1B), lambda b, j: (b, 0, j)),
            pl.BlockSpec((1, 1, S1B), lambda b, j: (b, 0, j)),
            pl.BlockSpec((1, 1, S1B), lambda b, j: (b, 0, j)),
        ],
        out_shape=[
            jax.ShapeDtypeStruct((B, D1, D2), jnp.bool_),
            jax.ShapeDtypeStruct((B, 1, D2), jnp.float32),
            jax.ShapeDtypeStruct((B, 1, D2), jnp.int32),
        ],
    )(x, thr)


def _sc_body(vmax_hbm, imax_hbm, thr_hbm, idx_hbm, cutv_hbm, cutd2_hbm,
             wq_hbm, dflag_hbm,
             vmax_v, imax_v, thr_v, key_a, d2_a, key_b, d2_b,
             hist_a, hist_b, digs_v, pos_v, starts_v, tots_v, outidx_v,
             cutv_v, cutd2_v, wq_v, dflag_v):
    b = lax.axis_index("s") * 2 + lax.axis_index("c")
    pltpu.sync_copy(vmax_hbm.at[pl.ds(b * D2, D2)], vmax_v)
    pltpu.sync_copy(imax_hbm.at[pl.ds(b * D2, D2)], imax_v)
    pltpu.sync_copy(thr_hbm, thr_v)

    lanes = lax.iota(jnp.int32, 16)
    zeros16 = jnp.zeros((16,), jnp.int32)
    ones16 = jnp.ones((16,), jnp.int32)
    base_bits = plsc.bitcast(thr_v[...], jnp.int32)
    span = (ONE_BITS - base_bits) + 1

    CH = D2 // 16

    @plsc.parallel_loop(0, CH, unroll=4, carry=zeros16)
    def z0(t, z0c):
        gi = t * 16 + lanes
        v = plsc.load_gather(vmax_v, [gi])
        r = plsc.load_gather(imax_v, [gi])
        vb = plsc.bitcast(v, jnp.int32)
        nzm = v > 0.0
        zval = jnp.where(nzm, vb - base_bits + 1, 0)
        key = r * (1 << 22) + (span - zval)
        plsc.store_scatter(key_a, [gi], key)
        plsc.store_scatter(d2_a, [gi], gi)
        plsc.store_scatter(wq_v, [gi], zeros16)

        @pl.when(t < 128)
        def _():
            plsc.store_scatter(hist_a, [t * 16 + lanes], zeros16)
        return z0c + plsc.all_reduce_population_count(~nzm)

    def hist_pass(src_key, hist, shift):
        @plsc.parallel_loop(0, CH, unroll=4)
        def _(t):
            gi = lanes * CH + t
            k = plsc.load_gather(src_key, [gi])
            dig = (k >> shift) & MASK7
            plsc.addupdate_scatter(hist, [dig * 16 + lanes], ones16)
            plsc.store_scatter(digs_v, [gi], dig)

    def scan_pass(hist, other, save_layout):
        def sloop(dig, carry):
            hidx = dig * 16 + lanes
            h = plsc.load_gather(hist, [hidx])
            c = plsc.cumsum(h)
            tot = jnp.sum(h)
            plsc.store_scatter(hist, [hidx], carry + (c - h))
            if other is not None:
                plsc.store_scatter(other, [hidx], zeros16)
            if save_layout:
                m0 = lanes == 0
                digs = zeros16 + dig
                plsc.store_scatter(starts_v, [digs], carry, mask=m0)
                plsc.store_scatter(tots_v, [digs], zeros16 + tot, mask=m0)
            return carry + tot
        lax.fori_loop(0, 128, sloop, zeros16)

    def perm_pass(src_key, src_d2, dst_key, dst_d2, hist):
        def posloop(t, c):
            gi = lanes * CH + t
            dig = plsc.load_gather(digs_v, [gi])
            hidx = dig * 16 + lanes
            pos = plsc.load_gather(hist, [hidx])
            plsc.store_scatter(hist, [hidx], pos + 1)
            plsc.store_scatter(pos_v, [gi], pos)
            return c
        lax.fori_loop(0, CH, posloop, 0)

        @plsc.parallel_loop(0, CH, unroll=4)
        def _(t):
            gi = t * 16 + lanes
            k = plsc.load_gather(src_key, [gi])
            d = plsc.load_gather(src_d2, [gi])
            p = plsc.load_gather(pos_v, [gi])
            plsc.store_scatter(dst_key, [p], k)
            plsc.store_scatter(dst_d2, [p], d)

    hist_pass(key_a, hist_a, 0)
    scan_pass(hist_a, hist_b, False)
    perm_pass(key_a, d2_a, key_b, d2_b, hist_a)
    hist_pass(key_b, hist_b, 7)
    scan_pass(hist_b, hist_a, False)
    perm_pass(key_b, d2_b, key_a, d2_a, hist_b)
    hist_pass(key_a, hist_a, 14)
    scan_pass(hist_a, hist_b, False)
    perm_pass(key_a, d2_a, key_b, d2_b, hist_a)
    hist_pass(key_b, hist_b, 21)
    scan_pass(hist_b, None, True)
    perm_pass(key_b, d2_b, key_a, d2_a, hist_b)

    @plsc.parallel_loop(0, D1, unroll=2, carry=zeros16 + (1 << 30))
    def minnz(r, mn):
        rsp = zeros16 + r
        r2 = rsp * 2
        startr = plsc.load_gather(starts_v, [r2])
        cnt = plsc.load_gather(tots_v, [r2]) + plsc.load_gather(tots_v, [r2 + 1])
        nz = jnp.where(rsp == 0, cnt - z0, cnt)

        for kk in range(4):
            posrow = kk * 16 + lanes
            sel = posrow < nz
            gidx = jnp.minimum(startr + posrow, D2 - 1)
            dv = plsc.load_gather(d2_a, [gidx])
            plsc.store_scatter(outidx_v, [r * K + posrow], jnp.where(sel, dv, 0))
            plsc.store_scatter(wq_v, [jnp.where(sel, dv, 0)], ones16, mask=sel)

        g63 = jnp.minimum(startr + (K - 1), D2 - 1)
        kv63 = plsc.load_gather(key_a, [g63])
        dv63 = plsc.load_gather(d2_a, [g63])
        sel63 = nz > (K - 1)
        zval63 = span - (kv63 & ((1 << 22) - 1))
        v63 = jnp.where(sel63 & (zval63 > 0),
                        plsc.bitcast(zval63 - 1 + base_bits, jnp.float32), 0.0)
        m0 = lanes == 0
        plsc.store_scatter(cutv_v, [rsp], v63, mask=m0)
        plsc.store_scatter(cutd2_v, [rsp], jnp.where(sel63, dv63, 0), mask=m0)
        return jnp.minimum(mn, nz)

    @pl.when(jnp.min(minnz) < K)
    def _pad():
        def fixr(r, c):
            rsp = zeros16 + r
            r2 = rsp * 2
            cnt = plsc.load_gather(tots_v, [r2]) + plsc.load_gather(tots_v, [r2 + 1])
            nz = jnp.where(rsp == 0, cnt - z0, cnt)

            @pl.when(jnp.min(nz) < K)
            def _fix():
                def scan_t(t, cntz):
                    gi = t * 16 + lanes
                    v = plsc.load_gather(vmax_v, [gi])
                    rr = plsc.load_gather(imax_v, [gi])
                    isz = (rr != rsp) | (v <= 0.0)
                    pref = plsc.cumsum(isz.astype(jnp.int32))
                    slot = nz + cntz + pref - 1
                    selp = isz & (slot >= nz) & (slot < K)
                    slot_c = jnp.clip(slot, 0, K - 1)
                    plsc.store_scatter(outidx_v, [r * K + slot_c], gi, mask=selp)
                    return cntz + plsc.all_reduce_population_count(isz)
                lax.fori_loop(0, NT, scan_t, zeros16)
                last = plsc.load_gather(outidx_v, [zeros16 + (r * K + K - 1)])
                plsc.store_scatter(cutd2_v, [rsp], last, mask=lanes == 0)
            return c
        lax.fori_loop(0, D1, fixr, 0)

    pltpu.sync_copy(outidx_v, idx_hbm.at[pl.ds(b * (D1 * K), D1 * K)])
    pltpu.sync_copy(cutv_v, cutv_hbm.at[pl.ds(b * D1, D1)])
    pltpu.sync_copy(cutd2_v, cutd2_hbm.at[pl.ds(b * D1, D1)])
    pltpu.sync_copy(wq_v, wq_hbm.at[pl.ds(b * D2, D2)])
    dflag = jnp.where(jnp.min(minnz) < K, 1, 0) + zeros16
    plsc.store_scatter(dflag_v, [lanes], dflag)
    pltpu.sync_copy(dflag_v, dflag_hbm.at[pl.ds(b * 16, 16)])


def _stage2(vflat, iflat, thr16):
    mesh = plsc.VectorSubcoreMesh(core_axis_name="c", subcore_axis_name="s")
    return pl.kernel(
        _sc_body,
        out_type=[
            jax.ShapeDtypeStruct((B * D1 * K,), jnp.int32),
            jax.ShapeDtypeStruct((B * D1,), jnp.float32),
            jax.ShapeDtypeStruct((B * D1,), jnp.int32),
            jax.ShapeDtypeStruct((B * D2,), jnp.int32),
            jax.ShapeDtypeStruct((B * 16,), jnp.int32),
        ],
        mesh=mesh,
        compiler_params=pltpu.CompilerParams(needs_layout_passes=False),
        scratch_types=[
            pltpu.VMEM((D2,), jnp.float32),
            pltpu.VMEM((D2,), jnp.int32),
            pltpu.VMEM((16,), jnp.float32),
            pltpu.VMEM((D2,), jnp.int32),
            pltpu.VMEM((D2,), jnp.int32),
            pltpu.VMEM((D2,), jnp.int32),
            pltpu.VMEM((D2,), jnp.int32),
            pltpu.VMEM((2048,), jnp.int32),
            pltpu.VMEM((2048,), jnp.int32),
            pltpu.VMEM((D2,), jnp.int32),
            pltpu.VMEM((D2,), jnp.int32),
            pltpu.VMEM((128,), jnp.int32),
            pltpu.VMEM((128,), jnp.int32),
            pltpu.VMEM((D1 * K,), jnp.int32),
            pltpu.VMEM((D1,), jnp.float32),
            pltpu.VMEM((D1,), jnp.int32),
            pltpu.VMEM((D2,), jnp.int32),
            pltpu.VMEM((16,), jnp.int32),
        ],
    )(vflat, iflat, thr16)


S3B = D2


def _stage3_body(dflag_ref, vmax_ref, imax_ref, cutv_ref, cutd2_ref, wq_ref,
                 feat_ref):
    imax = imax_ref[0]
    wq = wq_ref[0]
    d1ids = lax.broadcasted_iota(jnp.int32, (D1, S3B), 0)
    win = (imax == d1ids) & (wq != 0)

    @pl.when(dflag_ref[0, 0, 0] == 0)
    def _common():
        feat_ref[0] = win.astype(jnp.float32)

    @pl.when(dflag_ref[0, 0, 0] != 0)
    def _deficient():
        vmax = vmax_ref[0]
        cutv = cutv_ref[0]
        cutd2 = cutd2_ref[0]
        d2ids = (lax.broadcasted_iota(jnp.int32, (D1, S3B), 1)
                 + pl.program_id(1) * S3B)
        zpad = (cutv == 0.0) & (d2ids <= cutd2) & ((imax != d1ids) | (vmax == 0.0))
        feat_ref[0] = (win | zpad).astype(jnp.float32)


def _stage3(dflags, vmax3, imax3, cutv, cutd2, wq):
    return pl.pallas_call(
        _stage3_body,
        grid=(B, D2 // S3B),
        in_specs=[
            pl.BlockSpec(memory_space=pltpu.SMEM, block_shape=(1, 1, 1),
                         index_map=lambda b, j: (b, 0, 0)),
            pl.BlockSpec((1, 1, S3B), lambda b, j: (b, 0, j)),
            pl.BlockSpec((1, 1, S3B), lambda b, j: (b, 0, j)),
            pl.BlockSpec((1, D1, 1), lambda b, j: (b, 0, 0)),
            pl.BlockSpec((1, D1, 1), lambda b, j: (b, 0, 0)),
            pl.BlockSpec((1, 1, S3B), lambda b, j: (b, 0, j)),
        ],
        out_specs=pl.BlockSpec((1, D1, S3B), lambda b, j: (b, 0, j)),
        out_shape=jax.ShapeDtypeStruct((B, D1, D2), jnp.float32),
    )(dflags, vmax3, imax3, cutv, cutd2, wq)


def kernel(x, threshold):
    valid, vmax3, imax3 = _stage1(x, threshold)
    thr16 = jnp.full((16,), threshold, dtype=jnp.float32)
    idx_flat, cutv_flat, cutd2_flat, wq_flat, dflag_flat = _stage2(
        vmax3.reshape(-1), imax3.reshape(-1), thr16)
    indices = idx_flat.reshape(B, D1, K)
    dflags = dflag_flat.reshape(B, 16)[:, :1].reshape(B, 1, 1)
    feat = _stage3(dflags, vmax3, imax3,
                   cutv_flat.reshape(B, D1, 1), cutd2_flat.reshape(B, D1, 1),
                   wq_flat.reshape(B, 1, D2))
    return (feat, indices, valid)


# speedup vs baseline: 1.4406x; 1.0064x over previous
import jax
import jax.numpy as jnp
from jax import lax
from jax.experimental import pallas as pl
from jax.experimental.pallas import tpu as pltpu, tpu_sc as plsc

B, D1, D2 = 32, 64, 8192
K = 64
NT = D2 // 16
MASK7 = 127
ONE_BITS = 0x3F800000


def _stage1_body(x_ref, thr_ref, valid_ref, vmax_ref, imax_ref):
    x = x_ref[0]
    thr = thr_ref[0, 0]
    valid = x >= thr
    xm = jnp.where(valid, x, 0.0)
    vmax = jnp.max(xm, axis=0, keepdims=True)
    hit = xm == vmax
    ids = lax.broadcasted_iota(jnp.int32, xm.shape, 0)
    imax = jnp.min(jnp.where(hit, ids, D1), axis=0, keepdims=True)
    valid_ref[0] = valid
    vmax_ref[0] = vmax
    imax_ref[0] = imax


S1B = D2


def _stage1(x, threshold):
    thr = threshold.reshape(1, 1)
    return pl.pallas_call(
        _stage1_body,
        grid=(B, D2 // S1B),
        in_specs=[
            pl.BlockSpec((1, D1, S1B), lambda b, j: (b, 0, j)),
            pl.BlockSpec((1, 1), lambda b, j: (0, 0)),
        ],
        out_specs=[
            pl.BlockSpec((1, D1, S1B), lambda b, j: (b, 0, j)),
            pl.BlockSpec((1, 1, S1B), lambda b, j: (b, 0, j)),
            pl.BlockSpec((1, 1, S1B), lambda b, j: (b, 0, j)),
        ],
        out_shape=[
            jax.ShapeDtypeStruct((B, D1, D2), jnp.bool_),
            jax.ShapeDtypeStruct((B, 1, D2), jnp.float32),
            jax.ShapeDtypeStruct((B, 1, D2), jnp.int32),
        ],
    )(x, thr)


def _sc_body(vmax_hbm, imax_hbm, thr_hbm, idx_hbm, cutv_hbm, cutd2_hbm,
             wq_hbm, dflag_hbm,
             vmax_v, imax_v, thr_v, key_a, d2_a, key_b, d2_b,
             hist_a, hist_b, digs_v, pos_v, starts_v, tots_v, outidx_v,
             cutv_v, cutd2_v, wq_v, dflag_v):
    b = lax.axis_index("s") * 2 + lax.axis_index("c")
    pltpu.sync_copy(vmax_hbm.at[pl.ds(b * D2, D2)], vmax_v)
    pltpu.sync_copy(imax_hbm.at[pl.ds(b * D2, D2)], imax_v)
    pltpu.sync_copy(thr_hbm, thr_v)

    lanes = lax.iota(jnp.int32, 16)
    zeros16 = jnp.zeros((16,), jnp.int32)
    ones16 = jnp.ones((16,), jnp.int32)
    base_bits = plsc.bitcast(thr_v[...], jnp.int32)
    span = (ONE_BITS - base_bits) + 1

    CH = D2 // 16

    @plsc.parallel_loop(0, CH, unroll=8, carry=zeros16)
    def z0(t, z0c):
        gi = t * 16 + lanes
        v = plsc.load_gather(vmax_v, [gi])
        r = plsc.load_gather(imax_v, [gi])
        vb = plsc.bitcast(v, jnp.int32)
        nzm = v > 0.0
        zval = jnp.where(nzm, vb - base_bits + 1, 0)
        key = r * (1 << 22) + (span - zval)
        plsc.store_scatter(key_a, [gi], key)
        plsc.store_scatter(d2_a, [gi], gi)
        plsc.store_scatter(wq_v, [gi], zeros16)

        @pl.when(t < 128)
        def _():
            plsc.store_scatter(hist_a, [t * 16 + lanes], zeros16)
        return z0c + plsc.all_reduce_population_count(~nzm)

    def hist_pass(src_key, hist, shift):
        @plsc.parallel_loop(0, CH, unroll=8)
        def _(t):
            gi = lanes * CH + t
            k = plsc.load_gather(src_key, [gi])
            dig = (k >> shift) & MASK7
            plsc.addupdate_scatter(hist, [dig * 16 + lanes], ones16)
            plsc.store_scatter(digs_v, [gi], dig)

    def scan_pass(hist, other, save_layout):
        def sloop(dig, carry):
            hidx = dig * 16 + lanes
            h = plsc.load_gather(hist, [hidx])
            c = plsc.cumsum(h)
            tot = jnp.sum(h)
            plsc.store_scatter(hist, [hidx], carry + (c - h))
            if other is not None:
                plsc.store_scatter(other, [hidx], zeros16)
            if save_layout:
                m0 = lanes == 0
                digs = zeros16 + dig
                plsc.store_scatter(starts_v, [digs], carry, mask=m0)
                plsc.store_scatter(tots_v, [digs], zeros16 + tot, mask=m0)
            return carry + tot
        lax.fori_loop(0, 128, sloop, zeros16)

    def perm_pass(src_key, src_d2, dst_key, dst_d2, hist):
        def posloop(t, c):
            gi = lanes * CH + t
            dig = plsc.load_gather(digs_v, [gi])
            hidx = dig * 16 + lanes
            pos = plsc.load_gather(hist, [hidx])
            plsc.store_scatter(hist, [hidx], pos + 1)
            plsc.store_scatter(pos_v, [gi], pos)
            return c
        lax.fori_loop(0, CH, posloop, 0)

        @plsc.parallel_loop(0, CH, unroll=8)
        def _(t):
            gi = t * 16 + lanes
            k = plsc.load_gather(src_key, [gi])
            d = plsc.load_gather(src_d2, [gi])
            p = plsc.load_gather(pos_v, [gi])
            plsc.store_scatter(dst_key, [p], k)
            plsc.store_scatter(dst_d2, [p], d)

    hist_pass(key_a, hist_a, 0)
    scan_pass(hist_a, hist_b, False)
    perm_pass(key_a, d2_a, key_b, d2_b, hist_a)
    hist_pass(key_b, hist_b, 7)
    scan_pass(hist_b, hist_a, False)
    perm_pass(key_b, d2_b, key_a, d2_a, hist_b)
    hist_pass(key_a, hist_a, 14)
    scan_pass(hist_a, hist_b, False)
    perm_pass(key_a, d2_a, key_b, d2_b, hist_a)
    hist_pass(key_b, hist_b, 21)
    scan_pass(hist_b, None, True)
    perm_pass(key_b, d2_b, key_a, d2_a, hist_b)

    @plsc.parallel_loop(0, D1, unroll=2, carry=zeros16 + (1 << 30))
    def minnz(r, mn):
        rsp = zeros16 + r
        r2 = rsp * 2
        startr = plsc.load_gather(starts_v, [r2])
        cnt = plsc.load_gather(tots_v, [r2]) + plsc.load_gather(tots_v, [r2 + 1])
        nz = jnp.where(rsp == 0, cnt - z0, cnt)

        for kk in range(4):
            posrow = kk * 16 + lanes
            sel = posrow < nz
            gidx = jnp.minimum(startr + posrow, D2 - 1)
            dv = plsc.load_gather(d2_a, [gidx])
            plsc.store_scatter(outidx_v, [r * K + posrow], jnp.where(sel, dv, 0))
            plsc.store_scatter(wq_v, [jnp.where(sel, dv, 0)], ones16, mask=sel)

        g63 = jnp.minimum(startr + (K - 1), D2 - 1)
        kv63 = plsc.load_gather(key_a, [g63])
        dv63 = plsc.load_gather(d2_a, [g63])
        sel63 = nz > (K - 1)
        zval63 = span - (kv63 & ((1 << 22) - 1))
        v63 = jnp.where(sel63 & (zval63 > 0),
                        plsc.bitcast(zval63 - 1 + base_bits, jnp.float32), 0.0)
        m0 = lanes == 0
        plsc.store_scatter(cutv_v, [rsp], v63, mask=m0)
        plsc.store_scatter(cutd2_v, [rsp], jnp.where(sel63, dv63, 0), mask=m0)
        return jnp.minimum(mn, nz)

    @pl.when(jnp.min(minnz) < K)
    def _pad():
        def fixr(r, c):
            rsp = zeros16 + r
            r2 = rsp * 2
            cnt = plsc.load_gather(tots_v, [r2]) + plsc.load_gather(tots_v, [r2 + 1])
            nz = jnp.where(rsp == 0, cnt - z0, cnt)

            @pl.when(jnp.min(nz) < K)
            def _fix():
                def scan_t(t, cntz):
                    gi = t * 16 + lanes
                    v = plsc.load_gather(vmax_v, [gi])
                    rr = plsc.load_gather(imax_v, [gi])
                    isz = (rr != rsp) | (v <= 0.0)
                    pref = plsc.cumsum(isz.astype(jnp.int32))
                    slot = nz + cntz + pref - 1
                    selp = isz & (slot >= nz) & (slot < K)
                    slot_c = jnp.clip(slot, 0, K - 1)
                    plsc.store_scatter(outidx_v, [r * K + slot_c], gi, mask=selp)
                    return cntz + plsc.all_reduce_population_count(isz)
                lax.fori_loop(0, NT, scan_t, zeros16)
                last = plsc.load_gather(outidx_v, [zeros16 + (r * K + K - 1)])
                plsc.store_scatter(cutd2_v, [rsp], last, mask=lanes == 0)
            return c
        lax.fori_loop(0, D1, fixr, 0)

    pltpu.sync_copy(outidx_v, idx_hbm.at[pl.ds(b * (D1 * K), D1 * K)])
    pltpu.sync_copy(cutv_v, cutv_hbm.at[pl.ds(b * D1, D1)])
    pltpu.sync_copy(cutd2_v, cutd2_hbm.at[pl.ds(b * D1, D1)])
    pltpu.sync_copy(wq_v, wq_hbm.at[pl.ds(b * D2, D2)])
    dflag = jnp.where(jnp.min(minnz) < K, 1, 0) + zeros16
    plsc.store_scatter(dflag_v, [lanes], dflag)
    pltpu.sync_copy(dflag_v, dflag_hbm.at[pl.ds(b * 16, 16)])


def _stage2(vflat, iflat, thr16):
    mesh = plsc.VectorSubcoreMesh(core_axis_name="c", subcore_axis_name="s")
    return pl.kernel(
        _sc_body,
        out_type=[
            jax.ShapeDtypeStruct((B * D1 * K,), jnp.int32),
            jax.ShapeDtypeStruct((B * D1,), jnp.float32),
            jax.ShapeDtypeStruct((B * D1,), jnp.int32),
            jax.ShapeDtypeStruct((B * D2,), jnp.int32),
            jax.ShapeDtypeStruct((B * 16,), jnp.int32),
        ],
        mesh=mesh,
        compiler_params=pltpu.CompilerParams(needs_layout_passes=False),
        scratch_types=[
            pltpu.VMEM((D2,), jnp.float32),
            pltpu.VMEM((D2,), jnp.int32),
            pltpu.VMEM((16,), jnp.float32),
            pltpu.VMEM((D2,), jnp.int32),
            pltpu.VMEM((D2,), jnp.int32),
            pltpu.VMEM((D2,), jnp.int32),
            pltpu.VMEM((D2,), jnp.int32),
            pltpu.VMEM((2048,), jnp.int32),
            pltpu.VMEM((2048,), jnp.int32),
            pltpu.VMEM((D2,), jnp.int32),
            pltpu.VMEM((D2,), jnp.int32),
            pltpu.VMEM((128,), jnp.int32),
            pltpu.VMEM((128,), jnp.int32),
            pltpu.VMEM((D1 * K,), jnp.int32),
            pltpu.VMEM((D1,), jnp.float32),
            pltpu.VMEM((D1,), jnp.int32),
            pltpu.VMEM((D2,), jnp.int32),
            pltpu.VMEM((16,), jnp.int32),
        ],
    )(vflat, iflat, thr16)


S3B = D2


def _stage3_body(dflag_ref, vmax_ref, imax_ref, cutv_ref, cutd2_ref, wq_ref,
                 feat_ref):
    imax = imax_ref[0]
    wq = wq_ref[0]
    d1ids = lax.broadcasted_iota(jnp.int32, (D1, S3B), 0)
    win = (imax == d1ids) & (wq != 0)

    @pl.when(dflag_ref[0, 0, 0] == 0)
    def _common():
        feat_ref[0] = win.astype(jnp.float32)

    @pl.when(dflag_ref[0, 0, 0] != 0)
    def _deficient():
        vmax = vmax_ref[0]
        cutv = cutv_ref[0]
        cutd2 = cutd2_ref[0]
        d2ids = (lax.broadcasted_iota(jnp.int32, (D1, S3B), 1)
                 + pl.program_id(1) * S3B)
        zpad = (cutv == 0.0) & (d2ids <= cutd2) & ((imax != d1ids) | (vmax == 0.0))
        feat_ref[0] = (win | zpad).astype(jnp.float32)


def _stage3(dflags, vmax3, imax3, cutv, cutd2, wq):
    return pl.pallas_call(
        _stage3_body,
        grid=(B, D2 // S3B),
        in_specs=[
            pl.BlockSpec(memory_space=pltpu.SMEM, block_shape=(1, 1, 1),
                         index_map=lambda b, j: (b, 0, 0)),
            pl.BlockSpec((1, 1, S3B), lambda b, j: (b, 0, j)),
            pl.BlockSpec((1, 1, S3B), lambda b, j: (b, 0, j)),
            pl.BlockSpec((1, D1, 1), lambda b, j: (b, 0, 0)),
            pl.BlockSpec((1, D1, 1), lambda b, j: (b, 0, 0)),
            pl.BlockSpec((1, 1, S3B), lambda b, j: (b, 0, j)),
        ],
        out_specs=pl.BlockSpec((1, D1, S3B), lambda b, j: (b, 0, j)),
        out_shape=jax.ShapeDtypeStruct((B, D1, D2), jnp.float32),
    )(dflags, vmax3, imax3, cutv, cutd2, wq)


def kernel(x, threshold):
    valid, vmax3, imax3 = _stage1(x, threshold)
    thr16 = jnp.full((16,), threshold, dtype=jnp.float32)
    idx_flat, cutv_flat, cutd2_flat, wq_flat, dflag_flat = _stage2(
        vmax3.reshape(-1), imax3.reshape(-1), thr16)
    indices = idx_flat.reshape(B, D1, K)
    dflags = dflag_flat.reshape(B, 16)[:, :1].reshape(B, 1, 1)
    feat = _stage3(dflags, vmax3, imax3,
                   cutv_flat.reshape(B, D1, 1), cutd2_flat.reshape(B, D1, 1),
                   wq_flat.reshape(B, 1, D2))
    return (feat, indices, valid)
